# speedup vs baseline: 4.1965x; 4.1965x over previous
import functools

import jax
import jax.numpy as jnp
from jax.experimental import pallas as pl

BATCH = 128
N_INPUT = 262144
N_MODULES = 256
H1 = 256
H2 = 128
N_OUT = 10
OUT_PAD = 128

_BLK = 2048
_NB = N_INPUT // _BLK


def _seg_body(x_ref, idx_ref, w_ref, out_ref):
    j = pl.program_id(0)
    xw = x_ref[...] * w_ref[0, 0, :][None, :]
    iv = idx_ref[0, 0, :]
    oh = (iv[:, None] == jax.lax.broadcasted_iota(jnp.int32, (_BLK, N_MODULES), 1)
          ).astype(jnp.float32)
    part = jax.lax.dot_general(
        xw, oh, (((1,), (0,)), ((), ())),
        preferred_element_type=jnp.float32,
        precision=jax.lax.Precision.HIGHEST)

    @pl.when(j == 0)
    def _():
        out_ref[...] = jnp.zeros_like(out_ref)

    out_ref[...] += part


def _segment_sum(x, idx, w):
    idx3 = idx.astype(jnp.int32).reshape(_NB, 1, _BLK)
    w3 = w.reshape(_NB, 1, _BLK)
    return pl.pallas_call(
        _seg_body,
        grid=(_NB,),
        in_specs=[
            pl.BlockSpec((BATCH, _BLK), lambda j: (0, j)),
            pl.BlockSpec((1, 1, _BLK), lambda j: (j, 0, 0)),
            pl.BlockSpec((1, 1, _BLK), lambda j: (j, 0, 0)),
        ],
        out_specs=pl.BlockSpec((BATCH, N_MODULES), lambda j: (0, 0)),
        out_shape=jax.ShapeDtypeStruct((BATCH, N_MODULES), jnp.float32),
    )(x, idx3, w3)


def _bn(h, gamma, beta):
    mu = jnp.mean(h, axis=0, keepdims=True)
    var = jnp.mean((h - mu) ** 2, axis=0, keepdims=True)
    return gamma * (h - mu) * jax.lax.rsqrt(var + 1e-5) + beta


def _dot(a, b):
    return jax.lax.dot_general(a, b, (((1,), (0,)), ((), ())),
                               preferred_element_type=jnp.float32,
                               precision=jax.lax.Precision.HIGHEST)


def _head_body(wx_ref, g0_ref, b0_ref, W1_ref, b1_ref, g1_ref, bb1_ref,
               W2_ref, b2_ref, g2_ref, bb2_ref, W3_ref, b3_ref,
               out_ref, z_ref):
    z = _bn(jnp.maximum(wx_ref[...], 0.0), g0_ref[...], b0_ref[...])
    z_ref[...] = z
    h = _bn(jnp.maximum(_dot(z, W1_ref[...]) + b1_ref[...], 0.0),
            g1_ref[...], bb1_ref[...])
    h = _bn(jnp.maximum(_dot(h, W2_ref[...]) + b2_ref[...], 0.0),
            g2_ref[...], bb2_ref[...])
    logits = _dot(h, W3_ref[...]) + b3_ref[...]
    col = jax.lax.broadcasted_iota(jnp.int32, (BATCH, OUT_PAD), 1)
    logits = jnp.where(col < N_OUT, logits, -1e30)
    m = jnp.max(logits, axis=-1, keepdims=True)
    e = jnp.exp(logits - m)
    out_ref[...] = e / jnp.sum(e, axis=-1, keepdims=True)


def _head(wx, g0, b0, W1, b1, g1, bb1, W2, b2, g2, bb2, W3, b3):
    W3p = jnp.zeros((H2, OUT_PAD), jnp.float32).at[:, :N_OUT].set(W3)
    b3p = jnp.zeros((1, OUT_PAD), jnp.float32).at[0, :N_OUT].set(b3)
    args = (wx, g0.reshape(1, -1), b0.reshape(1, -1), W1, b1.reshape(1, -1),
            g1.reshape(1, -1), bb1.reshape(1, -1), W2, b2.reshape(1, -1),
            g2.reshape(1, -1), bb2.reshape(1, -1), W3p, b3p)
    out, z = pl.pallas_call(
        _head_body,
        out_shape=(jax.ShapeDtypeStruct((BATCH, OUT_PAD), jnp.float32),
                   jax.ShapeDtypeStruct((BATCH, N_MODULES), jnp.float32)),
    )(*args)
    return out[:, :N_OUT], z


def kernel(x, idx, w, g0, b0, W1, b1, g1, bb1, W2, b2, g2, bb2, W3, b3):
    wx = _segment_sum(x, idx, w)
    out, z = _head(wx, g0, b0, W1, b1, g1, bb1, W2, b2, g2, bb2, W3, b3)
    return (out, z)
